# initial kernel scaffold (unmeasured)
import jax
import jax.numpy as jnp
from jax import lax
from jax.experimental import pallas as pl
from jax.experimental.pallas import tpu as pltpu

N_DEV = 4


def _make_ag_body(m, k_sh, n):

    def body(x_ref, w_ref, xg_ref, wg_ref,
             local_sems, sx_sems, rx_sems, sw_sems, rw_sems):
        me = lax.axis_index("i")

        barrier = pltpu.get_barrier_semaphore()
        for off in range(1, N_DEV):
            peer = lax.rem(me + off, N_DEV)
            pl.semaphore_signal(barrier, inc=1, device_id=(peer,),
                                device_id_type=pl.DeviceIdType.MESH)
        pl.semaphore_wait(barrier, N_DEV - 1)

        cx = pltpu.make_async_copy(
            x_ref, xg_ref.at[:, pl.ds(me * k_sh, k_sh)], local_sems.at[0])
        cw = pltpu.make_async_copy(
            w_ref, wg_ref.at[pl.ds(me * k_sh, k_sh), :], local_sems.at[1])
        cx.start()
        cw.start()

        sends = []
        for off in range(1, N_DEV):
            peer = lax.rem(me + off, N_DEV)
            sx = pltpu.make_async_remote_copy(
                src_ref=x_ref,
                dst_ref=xg_ref.at[:, pl.ds(me * k_sh, k_sh)],
                send_sem=sx_sems.at[off - 1],
                recv_sem=rx_sems.at[off - 1],
                device_id=(peer,),
                device_id_type=pl.DeviceIdType.MESH,
            )
            sw = pltpu.make_async_remote_copy(
                src_ref=w_ref,
                dst_ref=wg_ref.at[pl.ds(me * k_sh, k_sh), :],
                send_sem=sw_sems.at[off - 1],
                recv_sem=rw_sems.at[off - 1],
                device_id=(peer,),
                device_id_type=pl.DeviceIdType.MESH,
            )
            sx.start()
            sw.start()
            sends.append((sx, sw))

        for off in range(1, N_DEV):
            src = lax.rem(me - off + N_DEV, N_DEV)
            rx = pltpu.make_async_remote_copy(
                src_ref=x_ref,
                dst_ref=xg_ref.at[:, pl.ds(src * k_sh, k_sh)],
                send_sem=sx_sems.at[off - 1],
                recv_sem=rx_sems.at[off - 1],
                device_id=(me,),
                device_id_type=pl.DeviceIdType.MESH,
            )
            rx.wait_recv()
            rw = pltpu.make_async_remote_copy(
                src_ref=w_ref,
                dst_ref=wg_ref.at[pl.ds(src * k_sh, k_sh), :],
                send_sem=sw_sems.at[off - 1],
                recv_sem=rw_sems.at[off - 1],
                device_id=(me,),
                device_id_type=pl.DeviceIdType.MESH,
            )
            rw.wait_recv()

        for sx, sw in sends:
            sx.wait_send()
            sw.wait_send()
        cx.wait()
        cw.wait()

    return body


def _gemm_body(xg_ref, wg_ref, s_ref, o_ref):
    acc = lax.dot_general(
        xg_ref[...], wg_ref[...], (((1,), (0,)), ((), ())),
        preferred_element_type=jnp.float32)
    o_ref[...] = acc * s_ref[0, 0]


def kernel(x, w_mat, scale_x, scale_w):
    m, k_sh = x.shape
    _, n = w_mat.shape
    k = k_sh * N_DEV

    x8 = x.astype(jnp.float8_e5m2)
    w8 = w_mat.astype(jnp.float8_e5m2)

    xg, wg = pl.pallas_call(
        _make_ag_body(m, k_sh, n),
        out_shape=[
            jax.ShapeDtypeStruct((m, k), jnp.float8_e5m2),
            jax.ShapeDtypeStruct((k, n), jnp.float8_e5m2),
        ],
        in_specs=[
            pl.BlockSpec(memory_space=pltpu.MemorySpace.ANY),
            pl.BlockSpec(memory_space=pltpu.MemorySpace.ANY),
        ],
        out_specs=[
            pl.BlockSpec(memory_space=pltpu.MemorySpace.ANY),
            pl.BlockSpec(memory_space=pltpu.MemorySpace.ANY),
        ],
        scratch_shapes=[
            pltpu.SemaphoreType.DMA((2,)),
            pltpu.SemaphoreType.DMA((N_DEV - 1,)),
            pltpu.SemaphoreType.DMA((N_DEV - 1,)),
            pltpu.SemaphoreType.DMA((N_DEV - 1,)),
            pltpu.SemaphoreType.DMA((N_DEV - 1,)),
        ],
        compiler_params=pltpu.CompilerParams(collective_id=0),
    )(x8, w8)

    scale = (scale_x[0] * scale_w[0]).reshape(1, 1)

    bm, bn = 1024, 2048
    y = pl.pallas_call(
        _gemm_body,
        grid=(m // bm, n // bn),
        out_shape=jax.ShapeDtypeStruct((m, n), jnp.float32),
        in_specs=[
            pl.BlockSpec((bm, k), lambda i, j: (i, 0)),
            pl.BlockSpec((k, bn), lambda i, j: (0, j)),
            pl.BlockSpec((1, 1), lambda i, j: (0, 0),
                         memory_space=pltpu.SMEM),
        ],
        out_specs=pl.BlockSpec((bm, bn), lambda i, j: (i, j)),
        compiler_params=pltpu.CompilerParams(
            dimension_semantics=("parallel", "parallel")),
    )(xg, wg, scale)
    return y


# baseline (device time: 448760 ns/iter reference)
import jax
import jax.numpy as jnp
from jax import lax
from jax.experimental import pallas as pl
from jax.experimental.pallas import tpu as pltpu

N_DEV = 4


def _make_ag_body(m, k_sh, n):

    def body(x_ref, w_ref, xg_ref, wg_ref,
             local_sems, sx_sems, rx_sems, sw_sems, rw_sems):
        me = lax.axis_index("i")

        barrier = pltpu.get_barrier_semaphore()
        for off in range(1, N_DEV):
            peer = lax.rem(me + off, N_DEV)
            pl.semaphore_signal(barrier, inc=1, device_id=(peer,),
                                device_id_type=pl.DeviceIdType.MESH)
        pl.semaphore_wait(barrier, N_DEV - 1)

        cx = pltpu.make_async_copy(
            x_ref, xg_ref.at[:, pl.ds(me * k_sh, k_sh)], local_sems.at[0])
        cw = pltpu.make_async_copy(
            w_ref, wg_ref.at[pl.ds(me * k_sh, k_sh), :], local_sems.at[1])
        cx.start()
        cw.start()

        sends = []
        for off in range(1, N_DEV):
            peer = lax.rem(me + off, N_DEV)
            sx = pltpu.make_async_remote_copy(
                src_ref=x_ref,
                dst_ref=xg_ref.at[:, pl.ds(me * k_sh, k_sh)],
                send_sem=sx_sems.at[off - 1],
                recv_sem=rx_sems.at[off - 1],
                device_id=(peer,),
                device_id_type=pl.DeviceIdType.MESH,
            )
            sw = pltpu.make_async_remote_copy(
                src_ref=w_ref,
                dst_ref=wg_ref.at[pl.ds(me * k_sh, k_sh), :],
                send_sem=sw_sems.at[off - 1],
                recv_sem=rw_sems.at[off - 1],
                device_id=(peer,),
                device_id_type=pl.DeviceIdType.MESH,
            )
            sx.start()
            sw.start()
            sends.append((sx, sw))

        for off in range(1, N_DEV):
            src = lax.rem(me - off + N_DEV, N_DEV)
            rx = pltpu.make_async_remote_copy(
                src_ref=x_ref,
                dst_ref=xg_ref.at[:, pl.ds(src * k_sh, k_sh)],
                send_sem=sx_sems.at[off - 1],
                recv_sem=rx_sems.at[off - 1],
                device_id=(me,),
                device_id_type=pl.DeviceIdType.MESH,
            )
            rx.wait_recv()
            rw = pltpu.make_async_remote_copy(
                src_ref=w_ref,
                dst_ref=wg_ref.at[pl.ds(src * k_sh, k_sh), :],
                send_sem=sw_sems.at[off - 1],
                recv_sem=rw_sems.at[off - 1],
                device_id=(me,),
                device_id_type=pl.DeviceIdType.MESH,
            )
            rw.wait_recv()

        for sx, sw in sends:
            sx.wait_send()
            sw.wait_send()
        cx.wait()
        cw.wait()

    return body


def _gemm_body(xg_ref, wg_ref, s_ref, o_ref):
    acc = lax.dot_general(
        xg_ref[...], wg_ref[...], (((1,), (0,)), ((), ())),
        preferred_element_type=jnp.float32)
    o_ref[...] = acc * s_ref[0, 0]


def kernel(x, w_mat, scale_x, scale_w):
    m, k_sh = x.shape
    _, n = w_mat.shape
    k = k_sh * N_DEV

    x8 = x.astype(jnp.float8_e5m2)
    w8 = w_mat.astype(jnp.float8_e5m2)

    xg, wg = pl.pallas_call(
        _make_ag_body(m, k_sh, n),
        out_shape=[
            jax.ShapeDtypeStruct((m, k), jnp.float8_e5m2),
            jax.ShapeDtypeStruct((k, n), jnp.float8_e5m2),
        ],
        in_specs=[
            pl.BlockSpec(memory_space=pl.ANY),
            pl.BlockSpec(memory_space=pl.ANY),
        ],
        out_specs=[
            pl.BlockSpec(memory_space=pl.ANY),
            pl.BlockSpec(memory_space=pl.ANY),
        ],
        scratch_shapes=[
            pltpu.SemaphoreType.DMA((2,)),
            pltpu.SemaphoreType.DMA((N_DEV - 1,)),
            pltpu.SemaphoreType.DMA((N_DEV - 1,)),
            pltpu.SemaphoreType.DMA((N_DEV - 1,)),
            pltpu.SemaphoreType.DMA((N_DEV - 1,)),
        ],
        compiler_params=pltpu.CompilerParams(collective_id=0),
    )(x8, w8)

    scale = (scale_x[0] * scale_w[0]).reshape(1, 1)

    bm, bn = 1024, 2048
    y = pl.pallas_call(
        _gemm_body,
        grid=(m // bm, n // bn),
        out_shape=jax.ShapeDtypeStruct((m, n), jnp.float32),
        in_specs=[
            pl.BlockSpec((bm, k), lambda i, j: (i, 0)),
            pl.BlockSpec((k, bn), lambda i, j: (0, j)),
            pl.BlockSpec((1, 1), lambda i, j: (0, 0),
                         memory_space=pltpu.SMEM),
        ],
        out_specs=pl.BlockSpec((bm, bn), lambda i, j: (i, j)),
        compiler_params=pltpu.CompilerParams(
            dimension_semantics=("parallel", "parallel")),
    )(xg, wg, scale)
    return y


# device time: 381456 ns/iter; 1.1764x vs baseline; 1.1764x over previous
import jax
import jax.numpy as jnp
from jax import lax
from jax.experimental import pallas as pl
from jax.experimental.pallas import tpu as pltpu

N_DEV = 4


def _make_ag_body(m, k_sh, n):
    h = k_sh // 2

    def body(x_ref, w_ref, xg_ref, wg_ref,
             local_sems, sx_sems, rx_sems, sw_sems, rw_sems):
        me = lax.axis_index("i")
        a = jnp.bitwise_xor(me, 1)
        b = 3 - me

        def x_slot(q, half):
            return xg_ref.at[:, pl.ds((2 * q + half) * h, h)]

        def w_slot(q, half):
            return wg_ref.at[pl.ds((2 * q + half) * h, h), :]

        def x_half(half):
            return x_ref.at[:, pl.ds(half * h, h)]

        def w_half(half):
            return w_ref.at[pl.ds(half * h, h), :]

        def start(src, dst, ssem, rsem, dev):
            c = pltpu.make_async_remote_copy(
                src_ref=src, dst_ref=dst, send_sem=ssem, recv_sem=rsem,
                device_id=(dev,), device_id_type=pl.DeviceIdType.MESH)
            c.start()
            return c

        def recv_wait(dst, dummy_src, rsem):
            pltpu.make_async_remote_copy(
                src_ref=dummy_src, dst_ref=dst,
                send_sem=rsem, recv_sem=rsem,
                device_id=(me,),
                device_id_type=pl.DeviceIdType.MESH).wait_recv()

        barrier = pltpu.get_barrier_semaphore()
        for p in (a, b):
            pl.semaphore_signal(barrier, inc=1, device_id=(p,),
                                device_id_type=pl.DeviceIdType.MESH)
        pl.semaphore_wait(barrier, 2)

        cx = pltpu.make_async_copy(
            x_ref, xg_ref.at[:, pl.ds(me * k_sh, k_sh)], local_sems.at[0])
        cw = pltpu.make_async_copy(
            w_ref, wg_ref.at[pl.ds(me * k_sh, k_sh), :], local_sems.at[1])
        cx.start()
        cw.start()

        sends = []
        for s_sems, src, slot in ((sx_sems, x_half, x_slot),
                                  (sw_sems, w_half, w_slot)):
            r_sems = rx_sems if s_sems is sx_sems else rw_sems
            sends.append(start(src(0), slot(me, 0), s_sems.at[0], r_sems.at[0], a))
            sends.append(start(src(1), slot(me, 1), s_sems.at[1], r_sems.at[1], b))
            sends.append(start(src(0), slot(me, 0), s_sems.at[2], r_sems.at[2], b))
            sends.append(start(src(1), slot(me, 1), s_sems.at[4], r_sems.at[4], a))

        recv_wait(x_slot(a, 0), x_half(0), rx_sems.at[0])
        sends.append(start(x_slot(a, 0), x_slot(a, 0), sx_sems.at[3],
                           rx_sems.at[3], b))
        recv_wait(x_slot(b, 1), x_half(1), rx_sems.at[1])
        sends.append(start(x_slot(b, 1), x_slot(b, 1), sx_sems.at[5],
                           rx_sems.at[5], a))
        recv_wait(w_slot(a, 0), w_half(0), rw_sems.at[0])
        sends.append(start(w_slot(a, 0), w_slot(a, 0), sw_sems.at[3],
                           rw_sems.at[3], b))
        recv_wait(w_slot(b, 1), w_half(1), rw_sems.at[1])
        sends.append(start(w_slot(b, 1), w_slot(b, 1), sw_sems.at[5],
                           rw_sems.at[5], a))

        recv_wait(x_slot(b, 0), x_half(0), rx_sems.at[2])
        recv_wait(x_slot(jnp.bitwise_xor(b, 1), 0), x_half(0), rx_sems.at[3])
        recv_wait(x_slot(a, 1), x_half(1), rx_sems.at[4])
        recv_wait(x_slot(3 - a, 1), x_half(1), rx_sems.at[5])
        recv_wait(w_slot(b, 0), w_half(0), rw_sems.at[2])
        recv_wait(w_slot(jnp.bitwise_xor(b, 1), 0), w_half(0), rw_sems.at[3])
        recv_wait(w_slot(a, 1), w_half(1), rw_sems.at[4])
        recv_wait(w_slot(3 - a, 1), w_half(1), rw_sems.at[5])

        for c in sends:
            c.wait_send()
        cx.wait()
        cw.wait()

    return body


def _gemm_body(xg_ref, wg_ref, s_ref, o_ref):
    acc = lax.dot_general(
        xg_ref[...], wg_ref[...], (((1,), (0,)), ((), ())),
        preferred_element_type=jnp.float32)
    o_ref[...] = acc * s_ref[0, 0]


def kernel(x, w_mat, scale_x, scale_w):
    m, k_sh = x.shape
    _, n = w_mat.shape
    k = k_sh * N_DEV

    x8 = x.astype(jnp.float8_e5m2)
    w8 = w_mat.astype(jnp.float8_e5m2)

    xg, wg = pl.pallas_call(
        _make_ag_body(m, k_sh, n),
        out_shape=[
            jax.ShapeDtypeStruct((m, k), jnp.float8_e5m2),
            jax.ShapeDtypeStruct((k, n), jnp.float8_e5m2),
        ],
        in_specs=[
            pl.BlockSpec(memory_space=pl.ANY),
            pl.BlockSpec(memory_space=pl.ANY),
        ],
        out_specs=[
            pl.BlockSpec(memory_space=pl.ANY),
            pl.BlockSpec(memory_space=pl.ANY),
        ],
        scratch_shapes=[
            pltpu.SemaphoreType.DMA((2,)),
            pltpu.SemaphoreType.DMA((6,)),
            pltpu.SemaphoreType.DMA((6,)),
            pltpu.SemaphoreType.DMA((6,)),
            pltpu.SemaphoreType.DMA((6,)),
        ],
        compiler_params=pltpu.CompilerParams(collective_id=0),
    )(x8, w8)

    scale = (scale_x[0] * scale_w[0]).reshape(1, 1)

    bm, bn = 1024, 2048
    y = pl.pallas_call(
        _gemm_body,
        grid=(m // bm, n // bn),
        out_shape=jax.ShapeDtypeStruct((m, n), jnp.float32),
        in_specs=[
            pl.BlockSpec((bm, k), lambda i, j: (i, 0)),
            pl.BlockSpec((k, bn), lambda i, j: (0, j)),
            pl.BlockSpec((1, 1), lambda i, j: (0, 0),
                         memory_space=pltpu.SMEM),
        ],
        out_specs=pl.BlockSpec((bm, bn), lambda i, j: (i, j)),
        compiler_params=pltpu.CompilerParams(
            dimension_semantics=("parallel", "parallel")),
    )(xg, wg, scale)
    return y
